# baseline (speedup 1.0000x reference)
import functools

import jax
import jax.numpy as jnp
from jax import lax
from jax.experimental import pallas as pl
from jax.experimental.pallas import tpu as pltpu
from jax.experimental.pallas import tpu_sc as plsc

LANES = 16
NUM_CORES = 2
NUM_SUBCORES = 16
NW = NUM_CORES * NUM_SUBCORES


def _make_enmf_sc(n_per_w, d):
    mesh = plsc.VectorSubcoreMesh(core_axis_name="c", subcore_axis_name="s")

    @functools.partial(
        pl.kernel,
        mesh=mesh,
        compiler_params=pltpu.CompilerParams(
            needs_layout_passes=False, use_tc_tiling_on_sc=False),
        out_type=jax.ShapeDtypeStruct((NW * n_per_w,), jnp.float32),
        scratch_types=[
            pltpu.VMEM((n_per_w,), jnp.int32),
            pltpu.VMEM((n_per_w,), jnp.int32),
            pltpu.VMEM((d * n_per_w,), jnp.float32),
            pltpu.VMEM((d * n_per_w,), jnp.float32),
            pltpu.VMEM((d,), jnp.float32),
            pltpu.VMEM((n_per_w,), jnp.float32),
            pltpu.SemaphoreType.DMA,
        ],
    )
    def k(users_hbm, items_hbm, utt_hbm, itt_hbm, h_hbm, out_hbm,
          uidx_v, iidx_v, ucols_v, icols_v, h_v, out_v, sem):
        wid = lax.axis_index("s") * NUM_CORES + lax.axis_index("c")
        base = wid * n_per_w
        pltpu.sync_copy(users_hbm.at[pl.ds(base, n_per_w)], uidx_v)
        pltpu.sync_copy(items_hbm.at[pl.ds(base, n_per_w)], iidx_v)
        pltpu.sync_copy(h_hbm, h_v)

        copies = []
        for c in range(d):
            dst = pl.ds(c * n_per_w, n_per_w)
            copies.append(pltpu.async_copy(
                utt_hbm.at[c].at[uidx_v], ucols_v.at[dst], sem))
            copies.append(pltpu.async_copy(
                itt_hbm.at[c].at[iidx_v], icols_v.at[dst], sem))
        for cp in copies:
            cp.wait()

        hv = h_v[...]
        hs = [hv[c] for c in range(d)]

        def body(g, carry):
            sl = g * LANES
            acc = jnp.zeros((LANES,), jnp.float32)
            for c in range(d):
                u = ucols_v[pl.ds(c * n_per_w + sl, LANES)]
                i = icols_v[pl.ds(c * n_per_w + sl, LANES)]
                acc = acc + u * i * hs[c]
            out_v[pl.ds(sl, LANES)] = acc
            return carry

        lax.fori_loop(0, n_per_w // LANES, body, 0)
        pltpu.sync_copy(out_v, out_hbm.at[pl.ds(base, n_per_w)])

    return k


def kernel(users, items, user_table, item_table, h):
    n = users.shape[0]
    d = user_table.shape[1]
    n_per_w = n // NW
    out = _make_enmf_sc(n_per_w, d)(
        users, items, user_table.T, item_table.T, h)
    return out


# speedup vs baseline: 1.0010x; 1.0010x over previous
import functools

import jax
import jax.numpy as jnp
from jax import lax
from jax.experimental import pallas as pl
from jax.experimental.pallas import tpu as pltpu
from jax.experimental.pallas import tpu_sc as plsc

LANES = 16
NUM_CORES = 2
NUM_SUBCORES = 16
NW = NUM_CORES * NUM_SUBCORES


def _make_enmf_sc(n_per_w, d):
    mesh = plsc.VectorSubcoreMesh(core_axis_name="c", subcore_axis_name="s")

    @functools.partial(
        pl.kernel,
        mesh=mesh,
        compiler_params=pltpu.CompilerParams(
            needs_layout_passes=False, use_tc_tiling_on_sc=False),
        out_type=jax.ShapeDtypeStruct((NW * n_per_w,), jnp.float32),
        scratch_types=[
            pltpu.VMEM((n_per_w,), jnp.int32),
            pltpu.VMEM((n_per_w,), jnp.int32),
            pltpu.VMEM((d * n_per_w,), jnp.float32),
            pltpu.VMEM((d * n_per_w,), jnp.float32),
            pltpu.VMEM((d,), jnp.float32),
            pltpu.VMEM((n_per_w,), jnp.float32),
            pltpu.SemaphoreType.DMA,
        ],
    )
    def k(users_hbm, items_hbm, utt_hbm, itt_hbm, h_hbm, out_hbm,
          uidx_v, iidx_v, ucols_v, icols_v, h_v, out_v, sem):
        wid = lax.axis_index("s") * NUM_CORES + lax.axis_index("c")
        base = wid * n_per_w
        pltpu.sync_copy(users_hbm.at[pl.ds(base, n_per_w)], uidx_v)
        pltpu.sync_copy(items_hbm.at[pl.ds(base, n_per_w)], iidx_v)
        pltpu.sync_copy(h_hbm, h_v)

        n_ch = n_per_w // 128
        copies = []
        for c in range(d):
            for j in range(n_ch):
                src_sl = pl.ds(j * 128, 128)
                dst = pl.ds(c * n_per_w + j * 128, 128)
                copies.append(pltpu.async_copy(
                    utt_hbm.at[c].at[uidx_v.at[src_sl]], ucols_v.at[dst], sem))
                copies.append(pltpu.async_copy(
                    itt_hbm.at[c].at[iidx_v.at[src_sl]], icols_v.at[dst], sem))
        for cp in copies:
            cp.wait()

        hv = h_v[...]
        hs = [hv[c] for c in range(d)]

        def body(g, carry):
            sl = g * LANES
            acc = jnp.zeros((LANES,), jnp.float32)
            for c in range(d):
                u = ucols_v[pl.ds(c * n_per_w + sl, LANES)]
                i = icols_v[pl.ds(c * n_per_w + sl, LANES)]
                acc = acc + u * i * hs[c]
            out_v[pl.ds(sl, LANES)] = acc
            return carry

        lax.fori_loop(0, n_per_w // LANES, body, 0)
        pltpu.sync_copy(out_v, out_hbm.at[pl.ds(base, n_per_w)])

    return k


def kernel(users, items, user_table, item_table, h):
    n = users.shape[0]
    d = user_table.shape[1]
    n_per_w = n // NW
    out = _make_enmf_sc(n_per_w, d)(
        users, items, user_table.T, item_table.T, h)
    return out
